# Initial kernel scaffold; baseline (speedup 1.0000x reference)
import jax
import jax.numpy as jnp
from jax.experimental import pallas as pl


def kernel(x, gain, bias, log_mix, log_alpha, log_momentum, log_scale):
    raise NotImplementedError("write your pallas kernel here")


# speedup vs baseline: 9.6015x; 9.6015x over previous
import functools

import jax
import jax.numpy as jnp
from jax.experimental import pallas as pl
from jax.experimental.pallas import tpu as pltpu

_K_SIM = (4, 8, 16)
_K_CON = (2, 4, 8)
_NEG = -1e30
_POS = 1e30
_GUARD = 1e29


def _norms_body(x_ref, inv_ref):
    h = x_ref[0]
    d = h.shape[1]
    ones = jnp.ones((1, d), jnp.float32)
    n2 = jax.lax.dot_general(ones, h * h, (((1,), (1,)), ((), ())),
                             preferred_element_type=jnp.float32,
                             precision=jax.lax.Precision.HIGHEST)
    inv_ref[0] = 1.0 / jnp.maximum(jnp.sqrt(n2), 1e-12)


def _inv_norms(x):
    b, t, d = x.shape
    return pl.pallas_call(
        _norms_body,
        grid=(b,),
        in_specs=[pl.BlockSpec((1, t, d), lambda i: (i, 0, 0))],
        out_specs=pl.BlockSpec((1, 1, t), lambda i: (i, 0, 0)),
        out_shape=jax.ShapeDtypeStruct((b, 1, t), jnp.float32),
    )(x)


def _round_body(params_ref, gain_ref, bias_ref, inv_ref, hfull_ref,
                htile_ref, *rest, k_sim, k_con, tr, is_last):
    if is_last:
        x_ref, hout_ref, msgneg_s = rest
    else:
        hout_ref, invout_ref, msgneg_s = rest
    it = pl.program_id(1)
    nt = pl.num_programs(1)
    t = hfull_ref.shape[1]
    d = hfull_ref.shape[2]
    hi = htile_ref[0]
    h_all = hfull_ref[0]
    inv_all = inv_ref[0]
    row_base = it * tr

    s = jax.lax.dot_general(hi, h_all, (((1,), (1,)), ((), ())),
                            preferred_element_type=jnp.float32,
                            precision=jax.lax.Precision.HIGHEST)
    s = s * inv_all
    rows = row_base + jax.lax.broadcasted_iota(jnp.int32, (tr, t), 0)
    cols = jax.lax.broadcasted_iota(jnp.int32, (tr, t), 1)
    s = jnp.where(cols < rows, s, _NEG)

    a = jnp.zeros((tr, t), jnp.float32)
    for _ in range(k_sim):
        m = jnp.max(s, axis=1, keepdims=True)
        hit = jnp.logical_and(s == m, m > -_GUARD)
        first = jnp.min(jnp.where(hit, cols, t), axis=1, keepdims=True)
        sel = cols == first
        a = a + sel.astype(jnp.float32)
        s = jnp.where(sel, _NEG, s)

    rvec = row_base + jax.lax.broadcasted_iota(jnp.int32, (tr, 1), 0)
    deg = jnp.maximum(jnp.minimum(rvec, k_sim), 1).astype(jnp.float32)
    msg_pos = jax.lax.dot_general(a, h_all, (((1,), (0,)), ((), ())),
                                  preferred_element_type=jnp.float32,
                                  precision=jax.lax.Precision.HIGHEST) / deg

    msgneg_s[...] = jnp.zeros((tr, d), jnp.float32)
    if k_con > 0:
        @pl.when(it == nt - 1)
        def _():
            s2 = jnp.where(s <= -_GUARD, _POS, s)
            ac = jnp.zeros((tr, t), jnp.float32)
            s2c = s2
            for c in range(k_con):
                m = jnp.min(s2c, axis=1, keepdims=True)
                en = jnp.logical_and(rvec >= t - k_con + c + 1, m < _GUARD)
                hit = jnp.logical_and(s2c == m, en)
                first = jnp.min(jnp.where(hit, cols, t), axis=1,
                                keepdims=True)
                sel = cols == first
                ac = ac + sel.astype(jnp.float32)
                s2c = jnp.where(sel, _POS, s2c)
            degc = jnp.maximum(rvec - (t - k_con), 1).astype(jnp.float32)
            msgneg_s[...] = jax.lax.dot_general(
                ac, h_all, (((1,), (0,)), ((), ())),
                preferred_element_type=jnp.float32,
                precision=jax.lax.Precision.HIGHEST) / degc

    mix = params_ref[0]
    alpha = params_ref[1]
    momentum = params_ref[2]
    scale = params_ref[3]
    ctx = alpha * msg_pos + (1.0 - alpha) * msgneg_s[...]
    blended = mix * hi + (1.0 - mix) * ctx
    pre = blended * gain_ref[...] + bias_ref[...]
    hn = 0.5 * pre * (1.0 + jax.lax.erf(pre * 0.7071067811865476))
    ho = momentum * hi + (1.0 - momentum) * hn
    if is_last:
        hout_ref[0] = (ho - x_ref[0]) * scale
    else:
        hout_ref[0] = ho
        ones = jnp.ones((1, d), jnp.float32)
        n2 = jax.lax.dot_general(ones, ho * ho, (((1,), (1,)), ((), ())),
                                 preferred_element_type=jnp.float32,
                                 precision=jax.lax.Precision.HIGHEST)
        invout_ref[0] = 1.0 / jnp.maximum(jnp.sqrt(n2), 1e-12)


def _make_round(b, t, d, k_sim, k_con, tr, is_last):
    nt = t // tr
    in_specs = [
        pl.BlockSpec(memory_space=pltpu.SMEM),
        pl.BlockSpec((1, d), lambda bb, i: (0, 0)),
        pl.BlockSpec((1, d), lambda bb, i: (0, 0)),
        pl.BlockSpec((1, 1, t), lambda bb, i: (bb, 0, 0)),
        pl.BlockSpec((1, t, d), lambda bb, i: (bb, 0, 0)),
        pl.BlockSpec((1, tr, d), lambda bb, i: (bb, i, 0)),
    ]
    if is_last:
        in_specs.append(pl.BlockSpec((1, tr, d), lambda bb, i: (bb, i, 0)))
        out_shape = jax.ShapeDtypeStruct((b, t, d), jnp.float32)
        out_specs = pl.BlockSpec((1, tr, d), lambda bb, i: (bb, i, 0))
    else:
        out_shape = (jax.ShapeDtypeStruct((b, t, d), jnp.float32),
                     jax.ShapeDtypeStruct((b, 1, t), jnp.float32))
        out_specs = (pl.BlockSpec((1, tr, d), lambda bb, i: (bb, i, 0)),
                     pl.BlockSpec((1, 1, tr), lambda bb, i: (bb, 0, i)))
    return pl.pallas_call(
        functools.partial(_round_body, k_sim=k_sim, k_con=k_con, tr=tr,
                          is_last=is_last),
        grid=(b, nt),
        in_specs=in_specs,
        out_specs=out_specs,
        out_shape=out_shape,
        scratch_shapes=[pltpu.VMEM((tr, d), jnp.float32)],
    )


def kernel(x, gain, bias, log_mix, log_alpha, log_momentum, log_scale):
    b, t, d = x.shape
    r_total = gain.shape[0]
    momentum = jax.nn.sigmoid(log_momentum)
    scale = jax.nn.softplus(log_scale) + 0.01
    tr = min(256, t)

    inv = _inv_norms(x)
    h = x
    out = None
    for r in range(r_total):
        k_sim = min(_K_SIM[r], t - 1)
        k_con = min(_K_CON[r], max(0, t - 1 - k_sim))
        is_last = (r == r_total - 1)
        params = jnp.stack([jax.nn.sigmoid(log_mix[r]),
                            jax.nn.sigmoid(log_alpha[r]),
                            momentum, scale]).astype(jnp.float32)
        call = _make_round(b, t, d, k_sim, k_con, tr, is_last)
        if is_last:
            out = call(params, gain[r:r + 1], bias[r:r + 1], inv, h, h, x)
        else:
            h, inv = call(params, gain[r:r + 1], bias[r:r + 1], inv, h, h)
    return out


# speedup vs baseline: 14.4919x; 1.5093x over previous
import functools

import jax
import jax.numpy as jnp
from jax.experimental import pallas as pl
from jax.experimental.pallas import tpu as pltpu

_K_SIM = (4, 8, 16)
_K_CON = (2, 4, 8)
_NEG = -1e30
_POS = 1e30
_GUARD = 1e29


def _norms_body(x_ref, inv_ref):
    h = x_ref[0]
    d = h.shape[1]
    ones = jnp.ones((1, d), jnp.float32)
    n2 = jax.lax.dot_general(ones, h * h, (((1,), (1,)), ((), ())),
                             preferred_element_type=jnp.float32,
                             precision=jax.lax.Precision.HIGHEST)
    inv_ref[0] = 1.0 / jnp.maximum(jnp.sqrt(n2), 1e-12)


def _inv_norms(x):
    b, t, d = x.shape
    return pl.pallas_call(
        _norms_body,
        grid=(b,),
        in_specs=[pl.BlockSpec((1, t, d), lambda i: (i, 0, 0))],
        out_specs=pl.BlockSpec((1, 1, t), lambda i: (i, 0, 0)),
        out_shape=jax.ShapeDtypeStruct((b, 1, t), jnp.float32),
    )(x)


def _round_body(params_ref, gain_ref, bias_ref, inv_ref, hfull_ref,
                htile_ref, *rest, k_sim, k_con, tr, is_last):
    if is_last:
        x_ref, hout_ref, s_ref, msgneg_s = rest
    else:
        hout_ref, invout_ref, s_ref, msgneg_s = rest
    it = pl.program_id(1)
    nt = pl.num_programs(1)
    t = hfull_ref.shape[1]
    d = hfull_ref.shape[2]
    hi = htile_ref[0]
    h_all = hfull_ref[0]
    row_base = it * tr

    def _sim_tile(j, carry):
        off = pl.multiple_of(j * tr, tr)
        hc = hfull_ref[0, pl.ds(off, tr), :]
        st = jax.lax.dot_general(hi, hc, (((1,), (1,)), ((), ())),
                                 preferred_element_type=jnp.float32,
                                 precision=jax.lax.Precision.HIGHEST)
        st = st * inv_ref[0, :, pl.ds(off, tr)]
        lcols = off + jax.lax.broadcasted_iota(jnp.int32, (tr, tr), 1)
        lrows = row_base + jax.lax.broadcasted_iota(jnp.int32, (tr, tr), 0)
        st = jnp.where(lcols < lrows, st, _NEG)
        s_ref[:, pl.ds(off, tr)] = st
        return carry

    def _fill_tile(j, carry):
        off = pl.multiple_of(j * tr, tr)
        s_ref[:, pl.ds(off, tr)] = jnp.full((tr, tr), _NEG, jnp.float32)
        return carry

    jax.lax.fori_loop(0, it + 1, _sim_tile, 0)
    jax.lax.fori_loop(it + 1, nt, _fill_tile, 0)

    s = s_ref[...]
    for _ in range(k_sim):
        m = jnp.max(s, axis=1, keepdims=True)
        s = jnp.where(s == m, _NEG, s)
    rows = row_base + jax.lax.broadcasted_iota(jnp.int32, (tr, t), 0)
    cols = jax.lax.broadcasted_iota(jnp.int32, (tr, t), 1)
    a = jnp.where(jnp.logical_and(s <= -_GUARD, cols < rows), 1.0, 0.0)

    rvec = row_base + jax.lax.broadcasted_iota(jnp.int32, (tr, 1), 0)
    deg = jnp.maximum(jnp.minimum(rvec, k_sim), 1).astype(jnp.float32)
    msg_pos = jax.lax.dot_general(a, h_all, (((1,), (0,)), ((), ())),
                                  preferred_element_type=jnp.float32,
                                  precision=jax.lax.Precision.HIGHEST) / deg

    msgneg_s[...] = jnp.zeros((tr, d), jnp.float32)
    if k_con > 0:
        @pl.when(it == nt - 1)
        def _():
            sel_sim = s <= -_GUARD
            s2 = jnp.where(sel_sim, _POS, s)
            s2c = s2
            for c in range(k_con):
                m = jnp.min(s2c, axis=1, keepdims=True)
                en = rvec >= t - k_con + c + 1
                hit = jnp.logical_and(s2c == m, en)
                s2c = jnp.where(hit, _POS, s2c)
            ac = jnp.where(
                jnp.logical_and(s2c >= _GUARD,
                                jnp.logical_not(sel_sim)), 1.0, 0.0)
            degc = jnp.maximum(rvec - (t - k_con), 1).astype(jnp.float32)
            msgneg_s[...] = jax.lax.dot_general(
                ac, h_all, (((1,), (0,)), ((), ())),
                preferred_element_type=jnp.float32,
                precision=jax.lax.Precision.HIGHEST) / degc

    mix = params_ref[0]
    alpha = params_ref[1]
    momentum = params_ref[2]
    scale = params_ref[3]
    ctx = alpha * msg_pos + (1.0 - alpha) * msgneg_s[...]
    blended = mix * hi + (1.0 - mix) * ctx
    pre = blended * gain_ref[...] + bias_ref[...]
    hn = 0.5 * pre * (1.0 + jax.lax.erf(pre * 0.7071067811865476))
    ho = momentum * hi + (1.0 - momentum) * hn
    if is_last:
        hout_ref[0] = (ho - x_ref[0]) * scale
    else:
        hout_ref[0] = ho
        ones = jnp.ones((1, d), jnp.float32)
        n2 = jax.lax.dot_general(ones, ho * ho, (((1,), (1,)), ((), ())),
                                 preferred_element_type=jnp.float32,
                                 precision=jax.lax.Precision.HIGHEST)
        invout_ref[0] = 1.0 / jnp.maximum(jnp.sqrt(n2), 1e-12)


def _make_round(b, t, d, k_sim, k_con, tr, is_last):
    nt = t // tr
    in_specs = [
        pl.BlockSpec(memory_space=pltpu.SMEM),
        pl.BlockSpec((1, d), lambda bb, i: (0, 0)),
        pl.BlockSpec((1, d), lambda bb, i: (0, 0)),
        pl.BlockSpec((1, 1, t), lambda bb, i: (bb, 0, 0)),
        pl.BlockSpec((1, t, d), lambda bb, i: (bb, 0, 0)),
        pl.BlockSpec((1, tr, d), lambda bb, i: (bb, i, 0)),
    ]
    if is_last:
        in_specs.append(pl.BlockSpec((1, tr, d), lambda bb, i: (bb, i, 0)))
        out_shape = jax.ShapeDtypeStruct((b, t, d), jnp.float32)
        out_specs = pl.BlockSpec((1, tr, d), lambda bb, i: (bb, i, 0))
    else:
        out_shape = (jax.ShapeDtypeStruct((b, t, d), jnp.float32),
                     jax.ShapeDtypeStruct((b, 1, t), jnp.float32))
        out_specs = (pl.BlockSpec((1, tr, d), lambda bb, i: (bb, i, 0)),
                     pl.BlockSpec((1, 1, tr), lambda bb, i: (bb, 0, i)))
    return pl.pallas_call(
        functools.partial(_round_body, k_sim=k_sim, k_con=k_con, tr=tr,
                          is_last=is_last),
        grid=(b, nt),
        in_specs=in_specs,
        out_specs=out_specs,
        out_shape=out_shape,
        scratch_shapes=[pltpu.VMEM((tr, t), jnp.float32),
                        pltpu.VMEM((tr, d), jnp.float32)],
    )


def kernel(x, gain, bias, log_mix, log_alpha, log_momentum, log_scale):
    b, t, d = x.shape
    r_total = gain.shape[0]
    momentum = jax.nn.sigmoid(log_momentum)
    scale = jax.nn.softplus(log_scale) + 0.01
    tr = min(256, t)

    inv = _inv_norms(x)
    h = x
    out = None
    for r in range(r_total):
        k_sim = min(_K_SIM[r], t - 1)
        k_con = min(_K_CON[r], max(0, t - 1 - k_sim))
        is_last = (r == r_total - 1)
        params = jnp.stack([jax.nn.sigmoid(log_mix[r]),
                            jax.nn.sigmoid(log_alpha[r]),
                            momentum, scale]).astype(jnp.float32)
        call = _make_round(b, t, d, k_sim, k_con, tr, is_last)
        if is_last:
            out = call(params, gain[r:r + 1], bias[r:r + 1], inv, h, h, x)
        else:
            h, inv = call(params, gain[r:r + 1], bias[r:r + 1], inv, h, h)
    return out


# speedup vs baseline: 16.2432x; 1.1208x over previous
import functools

import jax
import jax.numpy as jnp
from jax.experimental import pallas as pl
from jax.experimental.pallas import tpu as pltpu

_K_SIM = (4, 8, 16)
_K_CON = (2, 4, 8)
_NEG = -1e30
_POS = 1e30
_GUARD = 1e29


def _norms_body(x_ref, inv_ref):
    h = x_ref[0]
    d = h.shape[1]
    ones = jnp.ones((1, d), jnp.float32)
    n2 = jax.lax.dot_general(ones, h * h, (((1,), (1,)), ((), ())),
                             preferred_element_type=jnp.float32,
                             precision=jax.lax.Precision.HIGHEST)
    inv_ref[0] = 1.0 / jnp.maximum(jnp.sqrt(n2), 1e-12)


def _inv_norms(x):
    b, t, d = x.shape
    return pl.pallas_call(
        _norms_body,
        grid=(b,),
        in_specs=[pl.BlockSpec((1, t, d), lambda i: (i, 0, 0))],
        out_specs=pl.BlockSpec((1, 1, t), lambda i: (i, 0, 0)),
        out_shape=jax.ShapeDtypeStruct((b, 1, t), jnp.float32),
    )(x)


def _round_body(params_ref, gain_ref, bias_ref, inv_ref, hfull_ref,
                htile_ref, *rest, k_sim, k_con, tr, is_last):
    if is_last:
        x_ref, hout_ref, s_ref, msgpos_s, msgneg_s = rest
    else:
        hout_ref, invout_ref, s_ref, msgpos_s, msgneg_s = rest
    it = pl.program_id(1)
    nt = pl.num_programs(1)
    t = hfull_ref.shape[1]
    d = hfull_ref.shape[2]
    hi = htile_ref[0]
    h_all = hfull_ref[0]
    row_base = it * tr

    def _sim_tile(j, carry):
        off = pl.multiple_of(j * tr, tr)
        hc = hfull_ref[0, pl.ds(off, tr), :]
        st = jax.lax.dot_general(hi, hc, (((1,), (1,)), ((), ())),
                                 preferred_element_type=jnp.float32,
                                 precision=jax.lax.Precision.HIGHEST)
        st = st * inv_ref[0, :, pl.ds(off, tr)]
        lcols = off + jax.lax.broadcasted_iota(jnp.int32, (tr, tr), 1)
        lrows = row_base + jax.lax.broadcasted_iota(jnp.int32, (tr, tr), 0)
        st = jnp.where(lcols < lrows, st, _NEG)
        s_ref[:, pl.ds(off, tr)] = st
        return carry

    def _fill_tile(j, carry):
        off = pl.multiple_of(j * tr, tr)
        s_ref[:, pl.ds(off, tr)] = jnp.full((tr, tr), _NEG, jnp.float32)
        return carry

    jax.lax.fori_loop(0, it + 1, _sim_tile, 0)

    rvec = row_base + jax.lax.broadcasted_iota(jnp.int32, (tr, 1), 0)
    deg = jnp.maximum(jnp.minimum(rvec, k_sim), 1).astype(jnp.float32)

    def _select_agg(width):
        s = s_ref[:, :width]
        for _ in range(k_sim):
            m = jnp.max(s, axis=1, keepdims=True)
            s = jnp.where(s == m, _NEG, s)
        rows_w = row_base + jax.lax.broadcasted_iota(jnp.int32, (tr, width), 0)
        cols_w = jax.lax.broadcasted_iota(jnp.int32, (tr, width), 1)
        a = jnp.where(jnp.logical_and(s <= -_GUARD, cols_w < rows_w),
                      1.0, 0.0)
        msg = jax.lax.dot_general(a, hfull_ref[0, :width, :],
                                  (((1,), (0,)), ((), ())),
                                  preferred_element_type=jnp.float32,
                                  precision=jax.lax.Precision.HIGHEST)
        msgpos_s[...] = msg / deg
        return s

    half = (nt // 2) * tr
    msgneg_s[...] = jnp.zeros((tr, d), jnp.float32)

    if half > 0:
        @pl.when(it < nt // 2)
        def _():
            jax.lax.fori_loop(it + 1, nt // 2, _fill_tile, 0)
            _select_agg(half)

    @pl.when(it >= nt // 2)
    def _():
        jax.lax.fori_loop(it + 1, nt, _fill_tile, 0)
        s = _select_agg(t)
        if k_con > 0:
            @pl.when(it == nt - 1)
            def _():
                sel_sim = s <= -_GUARD
                s2c = jnp.where(sel_sim, _POS, s)
                for c in range(k_con):
                    m = jnp.min(s2c, axis=1, keepdims=True)
                    en = rvec >= t - k_con + c + 1
                    hit = jnp.logical_and(s2c == m, en)
                    s2c = jnp.where(hit, _POS, s2c)
                ac = jnp.where(
                    jnp.logical_and(s2c >= _GUARD,
                                    jnp.logical_not(sel_sim)), 1.0, 0.0)
                degc = jnp.maximum(rvec - (t - k_con), 1).astype(jnp.float32)
                msgneg_s[...] = jax.lax.dot_general(
                    ac, h_all, (((1,), (0,)), ((), ())),
                    preferred_element_type=jnp.float32,
                    precision=jax.lax.Precision.HIGHEST) / degc

    msg_pos = msgpos_s[...]

    mix = params_ref[0]
    alpha = params_ref[1]
    momentum = params_ref[2]
    scale = params_ref[3]
    ctx = alpha * msg_pos + (1.0 - alpha) * msgneg_s[...]
    blended = mix * hi + (1.0 - mix) * ctx
    pre = blended * gain_ref[...] + bias_ref[...]
    hn = 0.5 * pre * (1.0 + jax.lax.erf(pre * 0.7071067811865476))
    ho = momentum * hi + (1.0 - momentum) * hn
    if is_last:
        hout_ref[0] = (ho - x_ref[0]) * scale
    else:
        hout_ref[0] = ho
        ones = jnp.ones((1, d), jnp.float32)
        n2 = jax.lax.dot_general(ones, ho * ho, (((1,), (1,)), ((), ())),
                                 preferred_element_type=jnp.float32,
                                 precision=jax.lax.Precision.HIGHEST)
        invout_ref[0] = 1.0 / jnp.maximum(jnp.sqrt(n2), 1e-12)


def _make_round(b, t, d, k_sim, k_con, tr, is_last):
    nt = t // tr
    in_specs = [
        pl.BlockSpec(memory_space=pltpu.SMEM),
        pl.BlockSpec((1, d), lambda bb, i: (0, 0)),
        pl.BlockSpec((1, d), lambda bb, i: (0, 0)),
        pl.BlockSpec((1, 1, t), lambda bb, i: (bb, 0, 0)),
        pl.BlockSpec((1, t, d), lambda bb, i: (bb, 0, 0)),
        pl.BlockSpec((1, tr, d), lambda bb, i: (bb, i, 0)),
    ]
    if is_last:
        in_specs.append(pl.BlockSpec((1, tr, d), lambda bb, i: (bb, i, 0)))
        out_shape = jax.ShapeDtypeStruct((b, t, d), jnp.float32)
        out_specs = pl.BlockSpec((1, tr, d), lambda bb, i: (bb, i, 0))
    else:
        out_shape = (jax.ShapeDtypeStruct((b, t, d), jnp.float32),
                     jax.ShapeDtypeStruct((b, 1, t), jnp.float32))
        out_specs = (pl.BlockSpec((1, tr, d), lambda bb, i: (bb, i, 0)),
                     pl.BlockSpec((1, 1, tr), lambda bb, i: (bb, 0, i)))
    return pl.pallas_call(
        functools.partial(_round_body, k_sim=k_sim, k_con=k_con, tr=tr,
                          is_last=is_last),
        grid=(b, nt),
        in_specs=in_specs,
        out_specs=out_specs,
        out_shape=out_shape,
        scratch_shapes=[pltpu.VMEM((tr, t), jnp.float32),
                        pltpu.VMEM((tr, d), jnp.float32),
                        pltpu.VMEM((tr, d), jnp.float32)],
    )


def kernel(x, gain, bias, log_mix, log_alpha, log_momentum, log_scale):
    b, t, d = x.shape
    r_total = gain.shape[0]
    momentum = jax.nn.sigmoid(log_momentum)
    scale = jax.nn.softplus(log_scale) + 0.01
    tr = min(256, t)

    inv = _inv_norms(x)
    h = x
    out = None
    for r in range(r_total):
        k_sim = min(_K_SIM[r], t - 1)
        k_con = min(_K_CON[r], max(0, t - 1 - k_sim))
        is_last = (r == r_total - 1)
        params = jnp.stack([jax.nn.sigmoid(log_mix[r]),
                            jax.nn.sigmoid(log_alpha[r]),
                            momentum, scale]).astype(jnp.float32)
        call = _make_round(b, t, d, k_sim, k_con, tr, is_last)
        if is_last:
            out = call(params, gain[r:r + 1], bias[r:r + 1], inv, h, h, x)
        else:
            h, inv = call(params, gain[r:r + 1], bias[r:r + 1], inv, h, h)
    return out


# speedup vs baseline: 21.0759x; 1.2975x over previous
import functools

import jax
import jax.numpy as jnp
from jax.experimental import pallas as pl
from jax.experimental.pallas import tpu as pltpu

_K_SIM = (4, 8, 16)
_K_CON = (2, 4, 8)
_NEG = -1e30
_POS = 1e30
_GUARD = 1e29


def _norms_body(x_ref, inv_ref, x1_ref, x2_ref):
    h = x_ref[0]
    d = h.shape[1]
    ones = jnp.ones((1, d), jnp.float32)
    n2 = jax.lax.dot_general(ones, h * h, (((1,), (1,)), ((), ())),
                             preferred_element_type=jnp.float32,
                             precision=jax.lax.Precision.HIGHEST)
    inv_ref[0] = 1.0 / jnp.maximum(jnp.sqrt(n2), 1e-12)
    h1 = h.astype(jnp.bfloat16)
    x1_ref[0] = h1
    x2_ref[0] = (h - h1.astype(jnp.float32)).astype(jnp.bfloat16)


def _prep(x):
    b, t, d = x.shape
    return pl.pallas_call(
        _norms_body,
        grid=(b,),
        in_specs=[pl.BlockSpec((1, t, d), lambda i: (i, 0, 0))],
        out_specs=(pl.BlockSpec((1, 1, t), lambda i: (i, 0, 0)),
                   pl.BlockSpec((1, t, d), lambda i: (i, 0, 0)),
                   pl.BlockSpec((1, t, d), lambda i: (i, 0, 0))),
        out_shape=(jax.ShapeDtypeStruct((b, 1, t), jnp.float32),
                   jax.ShapeDtypeStruct((b, t, d), jnp.bfloat16),
                   jax.ShapeDtypeStruct((b, t, d), jnp.bfloat16)),
    )(x)


def _round_body(params_ref, gain_ref, bias_ref, inv_ref, hfull_ref,
                h1full_ref, h2full_ref, htile_ref, *rest,
                k_sim, k_con, tr, is_last):
    if is_last:
        x_ref, hout_ref, s_ref, msgpos_s, msgneg_s = rest
    else:
        (hout_ref, h1out_ref, h2out_ref, invout_ref,
         s_ref, msgpos_s, msgneg_s) = rest
    it = pl.program_id(1)
    nt = pl.num_programs(1)
    t = hfull_ref.shape[1]
    d = hfull_ref.shape[2]
    hi = htile_ref[0]
    h_all = hfull_ref[0]
    row_base = it * tr

    def _sim_tile(j, carry):
        off = pl.multiple_of(j * tr, tr)
        hc = hfull_ref[0, pl.ds(off, tr), :]
        st = jax.lax.dot_general(hi, hc, (((1,), (1,)), ((), ())),
                                 preferred_element_type=jnp.float32,
                                 precision=jax.lax.Precision.HIGHEST)
        st = st * inv_ref[0, :, pl.ds(off, tr)]
        lcols = off + jax.lax.broadcasted_iota(jnp.int32, (tr, tr), 1)
        lrows = row_base + jax.lax.broadcasted_iota(jnp.int32, (tr, tr), 0)
        st = jnp.where(lcols < lrows, st, _NEG)
        s_ref[:, pl.ds(off, tr)] = st
        return carry

    def _fill_tile(j, carry):
        off = pl.multiple_of(j * tr, tr)
        s_ref[:, pl.ds(off, tr)] = jnp.full((tr, tr), _NEG, jnp.float32)
        return carry

    jax.lax.fori_loop(0, it + 1, _sim_tile, 0)

    rvec = row_base + jax.lax.broadcasted_iota(jnp.int32, (tr, 1), 0)
    deg = jnp.maximum(jnp.minimum(rvec, k_sim), 1).astype(jnp.float32)

    def _select_agg(width):
        s = s_ref[:, :width]
        for _ in range(k_sim):
            m = jnp.max(s, axis=1, keepdims=True)
            s = jnp.where(s == m, _NEG, s)
        rows_w = row_base + jax.lax.broadcasted_iota(jnp.int32, (tr, width), 0)
        cols_w = jax.lax.broadcasted_iota(jnp.int32, (tr, width), 1)
        a = jnp.where(jnp.logical_and(s <= -_GUARD, cols_w < rows_w),
                      1.0, 0.0).astype(jnp.bfloat16)
        msg = (jax.lax.dot_general(a, h1full_ref[0, :width, :],
                                   (((1,), (0,)), ((), ())),
                                   preferred_element_type=jnp.float32)
               + jax.lax.dot_general(a, h2full_ref[0, :width, :],
                                     (((1,), (0,)), ((), ())),
                                     preferred_element_type=jnp.float32))
        msgpos_s[...] = msg / deg
        return s

    half = (nt // 2) * tr
    msgneg_s[...] = jnp.zeros((tr, d), jnp.float32)

    if half > 0:
        @pl.when(it < nt // 2)
        def _():
            jax.lax.fori_loop(it + 1, nt // 2, _fill_tile, 0)
            _select_agg(half)

    @pl.when(it >= nt // 2)
    def _():
        jax.lax.fori_loop(it + 1, nt, _fill_tile, 0)
        s = _select_agg(t)
        if k_con > 0:
            @pl.when(it == nt - 1)
            def _():
                sel_sim = s <= -_GUARD
                s2c = jnp.where(sel_sim, _POS, s)
                for c in range(k_con):
                    m = jnp.min(s2c, axis=1, keepdims=True)
                    en = rvec >= t - k_con + c + 1
                    hit = jnp.logical_and(s2c == m, en)
                    s2c = jnp.where(hit, _POS, s2c)
                ac = jnp.where(
                    jnp.logical_and(s2c >= _GUARD,
                                    jnp.logical_not(sel_sim)), 1.0, 0.0)
                degc = jnp.maximum(rvec - (t - k_con), 1).astype(jnp.float32)
                msgneg_s[...] = jax.lax.dot_general(
                    ac, h_all, (((1,), (0,)), ((), ())),
                    preferred_element_type=jnp.float32,
                    precision=jax.lax.Precision.HIGHEST) / degc

    msg_pos = msgpos_s[...]

    mix = params_ref[0]
    alpha = params_ref[1]
    momentum = params_ref[2]
    scale = params_ref[3]
    ctx = alpha * msg_pos + (1.0 - alpha) * msgneg_s[...]
    blended = mix * hi + (1.0 - mix) * ctx
    pre = blended * gain_ref[...] + bias_ref[...]
    hn = 0.5 * pre * (1.0 + jax.lax.erf(pre * 0.7071067811865476))
    ho = momentum * hi + (1.0 - momentum) * hn
    if is_last:
        hout_ref[0] = (ho - x_ref[0]) * scale
    else:
        hout_ref[0] = ho
        ho1 = ho.astype(jnp.bfloat16)
        h1out_ref[0] = ho1
        h2out_ref[0] = (ho - ho1.astype(jnp.float32)).astype(jnp.bfloat16)
        ones = jnp.ones((1, d), jnp.float32)
        n2 = jax.lax.dot_general(ones, ho * ho, (((1,), (1,)), ((), ())),
                                 preferred_element_type=jnp.float32,
                                 precision=jax.lax.Precision.HIGHEST)
        invout_ref[0] = 1.0 / jnp.maximum(jnp.sqrt(n2), 1e-12)


def _make_round(b, t, d, k_sim, k_con, tr, is_last):
    nt = t // tr
    in_specs = [
        pl.BlockSpec(memory_space=pltpu.SMEM),
        pl.BlockSpec((1, d), lambda bb, i: (0, 0)),
        pl.BlockSpec((1, d), lambda bb, i: (0, 0)),
        pl.BlockSpec((1, 1, t), lambda bb, i: (bb, 0, 0)),
        pl.BlockSpec((1, t, d), lambda bb, i: (bb, 0, 0)),
        pl.BlockSpec((1, t, d), lambda bb, i: (bb, 0, 0)),
        pl.BlockSpec((1, t, d), lambda bb, i: (bb, 0, 0)),
        pl.BlockSpec((1, tr, d), lambda bb, i: (bb, i, 0)),
    ]
    if is_last:
        in_specs.append(pl.BlockSpec((1, tr, d), lambda bb, i: (bb, i, 0)))
        out_shape = jax.ShapeDtypeStruct((b, t, d), jnp.float32)
        out_specs = pl.BlockSpec((1, tr, d), lambda bb, i: (bb, i, 0))
    else:
        out_shape = (jax.ShapeDtypeStruct((b, t, d), jnp.float32),
                     jax.ShapeDtypeStruct((b, t, d), jnp.bfloat16),
                     jax.ShapeDtypeStruct((b, t, d), jnp.bfloat16),
                     jax.ShapeDtypeStruct((b, 1, t), jnp.float32))
        out_specs = (pl.BlockSpec((1, tr, d), lambda bb, i: (bb, i, 0)),
                     pl.BlockSpec((1, tr, d), lambda bb, i: (bb, i, 0)),
                     pl.BlockSpec((1, tr, d), lambda bb, i: (bb, i, 0)),
                     pl.BlockSpec((1, 1, tr), lambda bb, i: (bb, 0, i)))
    return pl.pallas_call(
        functools.partial(_round_body, k_sim=k_sim, k_con=k_con, tr=tr,
                          is_last=is_last),
        grid=(b, nt),
        in_specs=in_specs,
        out_specs=out_specs,
        out_shape=out_shape,
        scratch_shapes=[pltpu.VMEM((tr, t), jnp.float32),
                        pltpu.VMEM((tr, d), jnp.float32),
                        pltpu.VMEM((tr, d), jnp.float32)],
    )


def kernel(x, gain, bias, log_mix, log_alpha, log_momentum, log_scale):
    b, t, d = x.shape
    r_total = gain.shape[0]
    momentum = jax.nn.sigmoid(log_momentum)
    scale = jax.nn.softplus(log_scale) + 0.01
    tr = min(256, t)

    inv, h1, h2 = _prep(x)
    h = x
    out = None
    for r in range(r_total):
        k_sim = min(_K_SIM[r], t - 1)
        k_con = min(_K_CON[r], max(0, t - 1 - k_sim))
        is_last = (r == r_total - 1)
        params = jnp.stack([jax.nn.sigmoid(log_mix[r]),
                            jax.nn.sigmoid(log_alpha[r]),
                            momentum, scale]).astype(jnp.float32)
        call = _make_round(b, t, d, k_sim, k_con, tr, is_last)
        if is_last:
            out = call(params, gain[r:r + 1], bias[r:r + 1], inv,
                       h, h1, h2, h, x)
        else:
            h, h1, h2, inv = call(params, gain[r:r + 1], bias[r:r + 1], inv,
                                  h, h1, h2, h)
    return out


# speedup vs baseline: 21.5629x; 1.0231x over previous
import functools

import jax
import jax.numpy as jnp
from jax.experimental import pallas as pl
from jax.experimental.pallas import tpu as pltpu

_K_SIM = (4, 8, 16)
_K_CON = (2, 4, 8)
_NEG = -1e30
_POS = 1e30
_GUARD = 1e29


def _norms_body(x_ref, inv_ref, x1_ref, x2_ref):
    h = x_ref[0]
    d = h.shape[1]
    ones = jnp.ones((1, d), jnp.float32)
    n2 = jax.lax.dot_general(ones, h * h, (((1,), (1,)), ((), ())),
                             preferred_element_type=jnp.float32,
                             precision=jax.lax.Precision.HIGHEST)
    inv_ref[0] = 1.0 / jnp.maximum(jnp.sqrt(n2), 1e-12)
    h1 = h.astype(jnp.bfloat16)
    x1_ref[0] = h1
    x2_ref[0] = (h - h1.astype(jnp.float32)).astype(jnp.bfloat16)


def _prep(x):
    b, t, d = x.shape
    return pl.pallas_call(
        _norms_body,
        grid=(b,),
        in_specs=[pl.BlockSpec((1, t, d), lambda i: (i, 0, 0))],
        out_specs=(pl.BlockSpec((1, 1, t), lambda i: (i, 0, 0)),
                   pl.BlockSpec((1, t, d), lambda i: (i, 0, 0)),
                   pl.BlockSpec((1, t, d), lambda i: (i, 0, 0))),
        out_shape=(jax.ShapeDtypeStruct((b, 1, t), jnp.float32),
                   jax.ShapeDtypeStruct((b, t, d), jnp.bfloat16),
                   jax.ShapeDtypeStruct((b, t, d), jnp.bfloat16)),
    )(x)


def _round_body(params_ref, gain_ref, bias_ref, inv_ref, hfull_ref,
                h1full_ref, h2full_ref, htile_ref, *rest,
                k_sim, k_con, tr, is_last):
    if is_last:
        x_ref, hout_ref, s_ref, msgpos_s, msgneg_s = rest
    else:
        (hout_ref, h1out_ref, h2out_ref, invout_ref,
         s_ref, msgpos_s, msgneg_s) = rest
    it = pl.program_id(1)
    nt = pl.num_programs(1)
    t = hfull_ref.shape[1]
    d = hfull_ref.shape[2]
    hi = htile_ref[0]
    h_all = hfull_ref[0]
    row_base = it * tr

    def _sim_tile(j, carry):
        off = pl.multiple_of(j * tr, tr)
        hc = hfull_ref[0, pl.ds(off, tr), :]
        st = jax.lax.dot_general(hi, hc, (((1,), (1,)), ((), ())),
                                 preferred_element_type=jnp.float32,
                                 precision=jax.lax.Precision.HIGHEST)
        st = st * inv_ref[0, :, pl.ds(off, tr)]
        lcols = off + jax.lax.broadcasted_iota(jnp.int32, (tr, tr), 1)
        lrows = row_base + jax.lax.broadcasted_iota(jnp.int32, (tr, tr), 0)
        st = jnp.where(lcols < lrows, st, _NEG)
        s_ref[:, pl.ds(off, tr)] = st
        return carry

    def _fill_tile(j, carry):
        off = pl.multiple_of(j * tr, tr)
        s_ref[:, pl.ds(off, tr)] = jnp.full((tr, tr), _NEG, jnp.float32)
        return carry

    jax.lax.fori_loop(0, it + 1, _sim_tile, 0)

    rvec = row_base + jax.lax.broadcasted_iota(jnp.int32, (tr, 1), 0)
    deg = jnp.maximum(jnp.minimum(rvec, k_sim), 1).astype(jnp.float32)

    def _select_agg(width):
        s = s_ref[:, :width]
        for _ in range(k_sim):
            m = jnp.max(s, axis=1, keepdims=True)
            s = jnp.where(s == m, _NEG, s)
        rows_w = row_base + jax.lax.broadcasted_iota(jnp.int32, (tr, width), 0)
        cols_w = jax.lax.broadcasted_iota(jnp.int32, (tr, width), 1)
        a = jnp.where(jnp.logical_and(s <= -_GUARD, cols_w < rows_w),
                      1.0, 0.0).astype(jnp.bfloat16)
        msg = (jax.lax.dot_general(a, h1full_ref[0, :width, :],
                                   (((1,), (0,)), ((), ())),
                                   preferred_element_type=jnp.float32)
               + jax.lax.dot_general(a, h2full_ref[0, :width, :],
                                     (((1,), (0,)), ((), ())),
                                     preferred_element_type=jnp.float32))
        msgpos_s[...] = msg / deg
        return s

    def _con(s):
        if k_con == 0:
            return

        @pl.when(it == nt - 1)
        def _():
            sel_sim = s <= -_GUARD
            s2c = jnp.where(sel_sim, _POS, s)
            for c in range(k_con):
                m = jnp.min(s2c, axis=1, keepdims=True)
                en = rvec >= t - k_con + c + 1
                hit = jnp.logical_and(s2c == m, en)
                s2c = jnp.where(hit, _POS, s2c)
            ac = jnp.where(
                jnp.logical_and(s2c >= _GUARD,
                                jnp.logical_not(sel_sim)), 1.0, 0.0)
            degc = jnp.maximum(rvec - (t - k_con), 1).astype(jnp.float32)
            msgneg_s[...] = jax.lax.dot_general(
                ac, h_all, (((1,), (0,)), ((), ())),
                preferred_element_type=jnp.float32,
                precision=jax.lax.Precision.HIGHEST) / degc

    msgneg_s[...] = jnp.zeros((tr, d), jnp.float32)

    bounds = sorted({nt // 4, nt // 2, (3 * nt) // 4, nt})
    lb = 0
    for ub in bounds:
        if ub <= lb:
            continue

        @pl.when(jnp.logical_and(it >= lb, it < ub))
        def _(ub=ub):
            jax.lax.fori_loop(it + 1, ub, _fill_tile, 0)
            s = _select_agg(ub * tr)
            if ub == nt:
                _con(s)

        lb = ub

    msg_pos = msgpos_s[...]

    mix = params_ref[0]
    alpha = params_ref[1]
    momentum = params_ref[2]
    scale = params_ref[3]
    ctx = alpha * msg_pos + (1.0 - alpha) * msgneg_s[...]
    blended = mix * hi + (1.0 - mix) * ctx
    pre = blended * gain_ref[...] + bias_ref[...]
    hn = 0.5 * pre * (1.0 + jax.lax.erf(pre * 0.7071067811865476))
    ho = momentum * hi + (1.0 - momentum) * hn
    if is_last:
        hout_ref[0] = (ho - x_ref[0]) * scale
    else:
        hout_ref[0] = ho
        ho1 = ho.astype(jnp.bfloat16)
        h1out_ref[0] = ho1
        h2out_ref[0] = (ho - ho1.astype(jnp.float32)).astype(jnp.bfloat16)
        ones = jnp.ones((1, d), jnp.float32)
        n2 = jax.lax.dot_general(ones, ho * ho, (((1,), (1,)), ((), ())),
                                 preferred_element_type=jnp.float32,
                                 precision=jax.lax.Precision.HIGHEST)
        invout_ref[0] = 1.0 / jnp.maximum(jnp.sqrt(n2), 1e-12)


def _make_round(b, t, d, k_sim, k_con, tr, is_last):
    nt = t // tr
    in_specs = [
        pl.BlockSpec(memory_space=pltpu.SMEM),
        pl.BlockSpec((1, d), lambda bb, i: (0, 0)),
        pl.BlockSpec((1, d), lambda bb, i: (0, 0)),
        pl.BlockSpec((1, 1, t), lambda bb, i: (bb, 0, 0)),
        pl.BlockSpec((1, t, d), lambda bb, i: (bb, 0, 0)),
        pl.BlockSpec((1, t, d), lambda bb, i: (bb, 0, 0)),
        pl.BlockSpec((1, t, d), lambda bb, i: (bb, 0, 0)),
        pl.BlockSpec((1, tr, d), lambda bb, i: (bb, i, 0)),
    ]
    if is_last:
        in_specs.append(pl.BlockSpec((1, tr, d), lambda bb, i: (bb, i, 0)))
        out_shape = jax.ShapeDtypeStruct((b, t, d), jnp.float32)
        out_specs = pl.BlockSpec((1, tr, d), lambda bb, i: (bb, i, 0))
    else:
        out_shape = (jax.ShapeDtypeStruct((b, t, d), jnp.float32),
                     jax.ShapeDtypeStruct((b, t, d), jnp.bfloat16),
                     jax.ShapeDtypeStruct((b, t, d), jnp.bfloat16),
                     jax.ShapeDtypeStruct((b, 1, t), jnp.float32))
        out_specs = (pl.BlockSpec((1, tr, d), lambda bb, i: (bb, i, 0)),
                     pl.BlockSpec((1, tr, d), lambda bb, i: (bb, i, 0)),
                     pl.BlockSpec((1, tr, d), lambda bb, i: (bb, i, 0)),
                     pl.BlockSpec((1, 1, tr), lambda bb, i: (bb, 0, i)))
    return pl.pallas_call(
        functools.partial(_round_body, k_sim=k_sim, k_con=k_con, tr=tr,
                          is_last=is_last),
        grid=(b, nt),
        in_specs=in_specs,
        out_specs=out_specs,
        out_shape=out_shape,
        scratch_shapes=[pltpu.VMEM((tr, t), jnp.float32),
                        pltpu.VMEM((tr, d), jnp.float32),
                        pltpu.VMEM((tr, d), jnp.float32)],
    )


def kernel(x, gain, bias, log_mix, log_alpha, log_momentum, log_scale):
    b, t, d = x.shape
    r_total = gain.shape[0]
    momentum = jax.nn.sigmoid(log_momentum)
    scale = jax.nn.softplus(log_scale) + 0.01
    tr = min(256, t)

    inv, h1, h2 = _prep(x)
    h = x
    out = None
    for r in range(r_total):
        k_sim = min(_K_SIM[r], t - 1)
        k_con = min(_K_CON[r], max(0, t - 1 - k_sim))
        is_last = (r == r_total - 1)
        params = jnp.stack([jax.nn.sigmoid(log_mix[r]),
                            jax.nn.sigmoid(log_alpha[r]),
                            momentum, scale]).astype(jnp.float32)
        call = _make_round(b, t, d, k_sim, k_con, tr, is_last)
        if is_last:
            out = call(params, gain[r:r + 1], bias[r:r + 1], inv,
                       h, h1, h2, h, x)
        else:
            h, h1, h2, inv = call(params, gain[r:r + 1], bias[r:r + 1], inv,
                                  h, h1, h2, h)
    return out
